# baseline (speedup 1.0000x reference)
import functools

import jax
import jax.numpy as jnp
from jax import lax
from jax.experimental import pallas as pl
from jax.experimental.pallas import tpu as pltpu
from jax.experimental.pallas import tpu_sc as plsc

NVARS = 512
BATCH = 4096
L = 16
NVEC = NVARS // L


def _build_permute():
    info = plsc.get_sparse_core_info()
    nc, ns = info.num_cores, info.num_subcores
    nw = nc * ns
    rows_per_w = BATCH // nw
    chunk = 32
    n_chunks = rows_per_w // chunk
    chunk_elems = chunk * NVARS

    mesh = plsc.VectorSubcoreMesh(core_axis_name="c", subcore_axis_name="s")

    @functools.partial(
        pl.kernel,
        mesh=mesh,
        out_type=jax.ShapeDtypeStruct((BATCH * NVARS,), jnp.float32),
        compiler_params=pltpu.CompilerParams(needs_layout_passes=False),
        scratch_types=[
            pltpu.VMEM((NVARS,), jnp.int32),
            pltpu.VMEM((chunk_elems,), jnp.int32),
            pltpu.VMEM((chunk_elems,), jnp.float32),
            pltpu.VMEM((chunk_elems,), jnp.float32),
            pltpu.VMEM((chunk_elems,), jnp.float32),
            pltpu.VMEM((chunk_elems,), jnp.float32),
            pltpu.SemaphoreType.DMA,
            pltpu.SemaphoreType.DMA,
        ],
    )
    def permute(x_hbm, perm_hbm, out_hbm, idx_v, addr, in0, in1, out0, out1,
                in_sem, out_sem):
        wid = lax.axis_index("s") * nc + lax.axis_index("c")
        base = wid * rows_per_w * NVARS

        pltpu.sync_copy(perm_hbm, idx_v)
        cols = [idx_v[pl.ds(j * L, L)] for j in range(NVEC)]
        in_bufs = [in0, in1]
        out_bufs = [out0, out1]

        def start_in(c, slot):
            return pltpu.async_copy(
                x_hbm.at[pl.ds(base + c * chunk_elems, chunk_elems)],
                in_bufs[slot], in_sem)

        def start_out(c, slot):
            return pltpu.async_copy(
                out_bufs[slot],
                out_hbm.at[pl.ds(base + c * chunk_elems, chunk_elems)],
                out_sem)

        @plsc.parallel_loop(0, chunk, 1, unroll=4)
        def _build(r):
            rbase = jnp.full((L,), r * NVARS, jnp.int32)
            for j in range(NVEC):
                addr[pl.ds(r * NVARS + j * L, L)] = rbase + cols[j]

        def compute(slot):
            in_b = in_bufs[slot]
            out_b = out_bufs[slot]

            @plsc.parallel_loop(0, chunk * NVEC, 1, unroll=8)
            def _vec(v):
                idxv = addr[pl.ds(v * L, L)]
                g = plsc.load_gather(in_b, [idxv])
                out_b[pl.ds(v * L, L)] = g

        in_h = [None, None]
        out_h = [None, None]
        in_h[0] = start_in(0, 0)
        for c in range(n_chunks):
            slot = c % 2
            if c + 1 < n_chunks:
                in_h[1 - slot] = start_in(c + 1, 1 - slot)
            in_h[slot].wait()
            if out_h[slot] is not None:
                out_h[slot].wait()
            compute(slot)
            out_h[slot] = start_out(c, slot)
        for h in out_h:
            if h is not None:
                h.wait()

    return permute


_permute = _build_permute()


def kernel(x, context, perm):
    y_flat = _permute(x.reshape(-1), perm.astype(jnp.int32))
    return y_flat.reshape(BATCH, NVARS), 0


# speedup vs baseline: 1.5379x; 1.5379x over previous
import functools

import jax
import jax.numpy as jnp
from jax import lax
from jax.experimental import pallas as pl
from jax.experimental.pallas import tpu as pltpu
from jax.experimental.pallas import tpu_sc as plsc

NVARS = 512
BATCH = 4096
L = 16
NVEC = NVARS // L


def _build_permute():
    info = plsc.get_sparse_core_info()
    nc, ns = info.num_cores, info.num_subcores
    nw = nc * ns
    rows_per_w = BATCH // nw
    chunk = 32
    n_chunks = rows_per_w // chunk

    mesh = plsc.VectorSubcoreMesh(core_axis_name="c", subcore_axis_name="s")

    @functools.partial(
        pl.kernel,
        mesh=mesh,
        out_type=jax.ShapeDtypeStruct((BATCH, NVARS), jnp.float32),
        compiler_params=pltpu.CompilerParams(needs_layout_passes=False),
        scratch_types=[
            pltpu.VMEM((NVARS,), jnp.int32),
            pltpu.VMEM((chunk, NVARS), jnp.float32),
            pltpu.VMEM((chunk, NVARS), jnp.float32),
            pltpu.VMEM((chunk, NVARS), jnp.float32),
            pltpu.VMEM((chunk, NVARS), jnp.float32),
            pltpu.VMEM((chunk, NVARS), jnp.float32),
            pltpu.VMEM((chunk, NVARS), jnp.float32),
            pltpu.SemaphoreType.DMA,
            pltpu.SemaphoreType.DMA,
            pltpu.SemaphoreType.DMA,
        ],
    )
    def permute(x_hbm, perm_hbm, out_hbm, idx_v, in0, in1, in2, in3,
                out0, out1, idx_sem, in_sem, out_sem):
        wid = lax.axis_index("s") * nc + lax.axis_index("c")
        base = wid * rows_per_w

        in_bufs = [in0, in1, in2, in3]
        out_bufs = [out0, out1]

        idx_h = pltpu.async_copy(perm_hbm, idx_v, idx_sem)
        in_h = [
            pltpu.async_copy(
                x_hbm.at[pl.ds(base + c * chunk, chunk)], in_bufs[c], in_sem)
            for c in range(n_chunks)
        ]

        def start_out(c, slot):
            return pltpu.async_copy(
                out_bufs[slot],
                out_hbm.at[pl.ds(base + c * chunk, chunk)],
                out_sem)

        idx_h.wait()
        jgroup = 8

        def compute(c, slot):
            in_b = in_bufs[c]
            out_b = out_bufs[slot]

            for g in range(NVEC // jgroup):
                colsg = [idx_v[pl.ds((g * jgroup + jj) * L, L)]
                         for jj in range(jgroup)]

                @plsc.parallel_loop(0, chunk, 1, unroll=4)
                def _row(r, _colsg=colsg, _g=g):
                    rsplat = jnp.full((L,), r, jnp.int32)
                    for jj in range(jgroup):
                        gv = plsc.load_gather(in_b, [rsplat, _colsg[jj]])
                        out_b[r, pl.ds((_g * jgroup + jj) * L, L)] = gv

        out_h = [None, None]
        for c in range(n_chunks):
            slot = c % 2
            in_h[c].wait()
            if out_h[slot] is not None:
                out_h[slot].wait()
            compute(c, slot)
            out_h[slot] = start_out(c, slot)
        for h in out_h:
            if h is not None:
                h.wait()

    return permute


_permute = _build_permute()


def kernel(x, context, perm):
    y = _permute(x, perm.astype(jnp.int32))
    return y, 0
